# baseline (speedup 1.0000x reference)
import functools

import jax
import jax.numpy as jnp
from jax import lax
from jax.experimental import pallas as pl
from jax.experimental.pallas import tpu as pltpu
from jax.experimental.pallas import tpu_sc as plsc

B = 16384
D = 128
NC = 2
NS = 16
NW = NC * NS
BPW = B // NW
CH = 128
NCH = BPW // CH

@functools.lru_cache(maxsize=1)
def _build_gather_mul():
    mesh = plsc.VectorSubcoreMesh(core_axis_name="c", subcore_axis_name="s")

    @functools.partial(
        pl.kernel,
        mesh=mesh,
        out_type=jax.ShapeDtypeStruct((B, D), jnp.float32),
        scratch_types=[
            pltpu.VMEM((NCH, CH), jnp.int32),
            pltpu.VMEM((NCH, CH), jnp.int32),
            pltpu.VMEM((CH, D), jnp.float32),
            pltpu.VMEM((CH, D), jnp.float32),
            pltpu.SemaphoreType.DMA,
            pltpu.SemaphoreType.DMA,
        ],
    )
    def _gather_mul(s_hbm, t_hbm, u_hbm, v_hbm, out_hbm,
                    sidx, tidx, ubuf, vbuf, usem, vsem):
        wid = lax.axis_index("s") * NC + lax.axis_index("c")
        pltpu.sync_copy(s_hbm.at[wid], sidx)
        pltpu.sync_copy(t_hbm.at[wid], tidx)
        base = wid * BPW
        for j in range(NCH):
            cu = pltpu.async_copy(u_hbm.at[sidx.at[j]], ubuf, usem)
            cv = pltpu.async_copy(v_hbm.at[tidx.at[j]], vbuf, vsem)
            cu.wait()
            cv.wait()

            def body(r, carry):
                for c in range(D // 16):
                    sl = pl.ds(c * 16, 16)
                    ubuf[r, sl] = ubuf[r, sl] * vbuf[r, sl]
                return carry

            lax.fori_loop(0, CH, body, 0)
            pltpu.sync_copy(ubuf, out_hbm.at[pl.ds(base + j * CH, CH)])

    return _gather_mul


def _mlp_body(prod_ref, w1t_ref, b1_ref, wt1t_ref, bt1_ref, wpmit_ref,
              bpmi_ref, wt2t_ref, bt2_ref, wsignt_ref, bsign_ref,
              o1_ref, o2_ref):
    prod = prod_ref[...]
    edge = jnp.maximum(
        jnp.dot(prod, w1t_ref[...], preferred_element_type=jnp.float32)
        + b1_ref[...], 0.0)
    h1 = jnp.maximum(
        jnp.dot(edge, wt1t_ref[...], preferred_element_type=jnp.float32)
        + bt1_ref[...], 0.0)
    o1_ref[...] = (
        jnp.dot(h1, wpmit_ref[...], preferred_element_type=jnp.float32)
        + bpmi_ref[...])
    h2 = jnp.maximum(
        jnp.dot(edge, wt2t_ref[...], preferred_element_type=jnp.float32)
        + bt2_ref[...], 0.0)
    o2_ref[...] = jax.nn.sigmoid(
        jnp.dot(h2, wsignt_ref[...], preferred_element_type=jnp.float32)
        + bsign_ref[...])


def _pad_t(w, rows, cols):
    r0, c0 = w.shape
    wp = jnp.zeros((rows, cols), jnp.float32).at[:r0, :c0].set(w)
    return wp.T


def kernel(s, t, emb_u, emb_v, W1, b1, Wt1, bt1, Wpmi, bpmi, Wt2, bt2,
           Wsign, bsign):
    s2 = s.astype(jnp.int32).reshape(NW, NCH, CH)
    t2 = t.astype(jnp.int32).reshape(NW, NCH, CH)
    prod = _build_gather_mul()(s2, t2, emb_u, emb_v)

    w1t = _pad_t(W1, D, D)
    b1p = jnp.zeros((1, D), jnp.float32).at[0, :20].set(b1)
    wt1t = _pad_t(Wt1, D, D)
    bt1p = jnp.zeros((1, D), jnp.float32).at[0, :5].set(bt1)
    wpmit = _pad_t(Wpmi, D, D)
    bpmip = jnp.full((1, D), bpmi[0], jnp.float32)
    wt2t = _pad_t(Wt2, D, D)
    bt2p = jnp.zeros((1, D), jnp.float32).at[0, :5].set(bt2)
    wsignt = _pad_t(Wsign, D, D)
    bsignp = jnp.full((1, D), bsign[0], jnp.float32)

    blk = 2048
    grid = (B // blk,)
    row_spec = pl.BlockSpec((blk, D), lambda i: (i, 0))
    w_spec = pl.BlockSpec((D, D), lambda i: (0, 0))
    b_spec = pl.BlockSpec((1, D), lambda i: (0, 0))
    o1p, o2p = pl.pallas_call(
        _mlp_body,
        grid=grid,
        in_specs=[row_spec, w_spec, b_spec, w_spec, b_spec, w_spec, b_spec,
                  w_spec, b_spec, w_spec, b_spec],
        out_specs=[row_spec, row_spec],
        out_shape=[jax.ShapeDtypeStruct((B, D), jnp.float32),
                   jax.ShapeDtypeStruct((B, D), jnp.float32)],
    )(prod, w1t, b1p, wt1t, bt1p, wpmit, bpmip, wt2t, bt2p, wsignt, bsignp)
    return (o1p[:, :1], o2p[:, :1])


# speedup vs baseline: 1.0269x; 1.0269x over previous
import functools

import jax
import jax.numpy as jnp
from jax import lax
from jax.experimental import pallas as pl
from jax.experimental.pallas import tpu as pltpu
from jax.experimental.pallas import tpu_sc as plsc

B = 16384
D = 128
NC = 2
NS = 16
NW = NC * NS
BPW = B // NW
CH = 128
NCH = BPW // CH

@functools.lru_cache(maxsize=1)
def _build_gather_mul():
    mesh = plsc.VectorSubcoreMesh(core_axis_name="c", subcore_axis_name="s")

    @functools.partial(
        pl.kernel,
        mesh=mesh,
        out_type=jax.ShapeDtypeStruct((B, D), jnp.float32),
        scratch_types=[
            pltpu.VMEM((NCH, CH), jnp.int32),
            pltpu.VMEM((NCH, CH), jnp.int32),
            pltpu.VMEM((CH, D), jnp.float32),
            pltpu.VMEM((CH, D), jnp.float32),
            pltpu.SemaphoreType.DMA,
            pltpu.SemaphoreType.DMA,
        ],
    )
    def _gather_mul(s_hbm, t_hbm, u_hbm, v_hbm, out_hbm,
                    sidx, tidx, ubuf, vbuf, usem, vsem):
        wid = lax.axis_index("s") * NC + lax.axis_index("c")
        pltpu.sync_copy(s_hbm.at[wid], sidx)
        pltpu.sync_copy(t_hbm.at[wid], tidx)
        base = wid * BPW
        for j in range(NCH):
            cu = pltpu.async_copy(u_hbm.at[sidx.at[j]], ubuf, usem)
            cv = pltpu.async_copy(v_hbm.at[tidx.at[j]], vbuf, vsem)
            cu.wait()
            cv.wait()

            def body(r, carry):
                for c in range(D // 16):
                    sl = pl.ds(c * 16, 16)
                    ubuf[r, sl] = ubuf[r, sl] * vbuf[r, sl]
                return carry

            lax.fori_loop(0, CH, body, 0)
            pltpu.sync_copy(ubuf, out_hbm.at[pl.ds(base + j * CH, CH)])

    return _gather_mul


def _mlp_body(prod_ref, w1t_ref, b1_ref, wtt_ref, btt_ref, wps_ref,
              bps_ref, o1_ref, o2_ref):
    prod = prod_ref[...]
    edge = jnp.maximum(
        jnp.dot(prod, w1t_ref[...], preferred_element_type=jnp.float32)
        + b1_ref[...], 0.0)
    ht = jnp.maximum(
        jnp.dot(edge, wtt_ref[...], preferred_element_type=jnp.float32)
        + btt_ref[...], 0.0)
    o = (jnp.dot(ht, wps_ref[...], preferred_element_type=jnp.float32)
         + bps_ref[...])
    o1_ref[...] = o[:, 0:1]
    o2_ref[...] = jax.nn.sigmoid(o[:, 1:2])


def _pad_t(w, rows, cols):
    r0, c0 = w.shape
    wp = jnp.zeros((rows, cols), jnp.float32).at[:r0, :c0].set(w)
    return wp.T


def kernel(s, t, emb_u, emb_v, W1, b1, Wt1, bt1, Wpmi, bpmi, Wt2, bt2,
           Wsign, bsign):
    s2 = s.astype(jnp.int32).reshape(NW, NCH, CH)
    t2 = t.astype(jnp.int32).reshape(NW, NCH, CH)
    prod = _build_gather_mul()(s2, t2, emb_u, emb_v)

    w1t = _pad_t(W1, D, D)
    b1p = jnp.zeros((1, D), jnp.float32).at[0, :20].set(b1)
    wtt = (jnp.zeros((D, D), jnp.float32)
           .at[:20, 0:5].set(Wt1.T).at[:20, 5:10].set(Wt2.T))
    btt = (jnp.zeros((1, D), jnp.float32)
           .at[0, 0:5].set(bt1).at[0, 5:10].set(bt2))
    wps = (jnp.zeros((D, D), jnp.float32)
           .at[0:5, 0].set(Wpmi[0]).at[5:10, 1].set(Wsign[0]))
    bps = (jnp.zeros((1, D), jnp.float32)
           .at[0, 0].set(bpmi[0]).at[0, 1].set(bsign[0]))

    blk = 2048
    grid = (B // blk,)
    row_spec = pl.BlockSpec((blk, D), lambda i: (i, 0))
    w_spec = pl.BlockSpec((D, D), lambda i: (0, 0))
    b_spec = pl.BlockSpec((1, D), lambda i: (0, 0))
    o_spec = pl.BlockSpec((blk, 1), lambda i: (i, 0))
    out1, out2 = pl.pallas_call(
        _mlp_body,
        grid=grid,
        in_specs=[row_spec, w_spec, b_spec, w_spec, b_spec, w_spec, b_spec],
        out_specs=[o_spec, o_spec],
        out_shape=[jax.ShapeDtypeStruct((B, 1), jnp.float32),
                   jax.ShapeDtypeStruct((B, 1), jnp.float32)],
    )(prod, w1t, b1p, wtt, btt, wps, bps)
    return (out1, out2)


# speedup vs baseline: 1.3744x; 1.3384x over previous
import functools

import jax
import jax.numpy as jnp
from jax import lax
from jax.experimental import pallas as pl
from jax.experimental.pallas import tpu as pltpu
from jax.experimental.pallas import tpu_sc as plsc

B = 16384
D = 128
NC = 2
NS = 16
NW = NC * NS
BPW = B // NW
CH = 128
NCH = BPW // CH

@functools.lru_cache(maxsize=1)
def _build_gather_mul():
    mesh = plsc.VectorSubcoreMesh(core_axis_name="c", subcore_axis_name="s")

    @functools.partial(
        pl.kernel,
        mesh=mesh,
        out_type=jax.ShapeDtypeStruct((B, D), jnp.float32),
        scratch_types=[
            pltpu.VMEM((NCH, CH), jnp.int32),
            pltpu.VMEM((NCH, CH), jnp.int32),
            pltpu.VMEM((CH, D), jnp.float32),
            pltpu.VMEM((CH, D), jnp.float32),
            pltpu.SemaphoreType.DMA,
            pltpu.SemaphoreType.DMA,
        ],
    )
    def _gather_mul(s_hbm, t_hbm, u_hbm, v_hbm, out_hbm,
                    sidx, tidx, ubuf, vbuf, usem, vsem):
        wid = lax.axis_index("s") * NC + lax.axis_index("c")
        pltpu.sync_copy(s_hbm.at[wid], sidx)
        pltpu.sync_copy(t_hbm.at[wid], tidx)
        base = wid * BPW
        for j in range(NCH):
            cu = pltpu.async_copy(u_hbm.at[sidx.at[j]], ubuf, usem)
            cv = pltpu.async_copy(v_hbm.at[tidx.at[j]], vbuf, vsem)
            cu.wait()
            cv.wait()

            def body(r, carry):
                for c in range(D // 16):
                    sl = pl.ds(c * 16, 16)
                    ubuf[r, sl] = ubuf[r, sl] * vbuf[r, sl]
                return carry

            lax.fori_loop(0, CH, body, 0)
            pltpu.sync_copy(ubuf, out_hbm.at[pl.ds(base + j * CH, CH)])

    return _gather_mul


def _mlp_body(prod_ref, w1t_ref, b1_ref, wtt_ref, btt_ref, wpst_ref,
              bpmi_ref, bsign_ref, o1_ref, o2_ref):
    prod = prod_ref[...]
    edge = jnp.maximum(
        jnp.dot(prod, w1t_ref[...], preferred_element_type=jnp.float32)
        + b1_ref[...], 0.0)
    ht = jnp.maximum(
        jnp.dot(edge, wtt_ref[...], preferred_element_type=jnp.float32)
        + btt_ref[...], 0.0)
    ot = lax.dot_general(wpst_ref[...], ht, (((1,), (1,)), ((), ())),
                         preferred_element_type=jnp.float32)
    o1_ref[...] = ot[0, :] + bpmi_ref[0]
    o2_ref[...] = jax.nn.sigmoid(ot[1, :] + bsign_ref[0])


def _pad_t(w, rows, cols):
    r0, c0 = w.shape
    wp = jnp.zeros((rows, cols), jnp.float32).at[:r0, :c0].set(w)
    return wp.T


def kernel(s, t, emb_u, emb_v, W1, b1, Wt1, bt1, Wpmi, bpmi, Wt2, bt2,
           Wsign, bsign):
    s2 = s.astype(jnp.int32).reshape(NW, NCH, CH)
    t2 = t.astype(jnp.int32).reshape(NW, NCH, CH)
    prod = _build_gather_mul()(s2, t2, emb_u, emb_v)

    w1t = _pad_t(W1, D, D)
    b1p = jnp.zeros((1, D), jnp.float32).at[0, :20].set(b1)
    wtt = (jnp.zeros((D, D), jnp.float32)
           .at[:20, 0:5].set(Wt1.T).at[:20, 5:10].set(Wt2.T))
    btt = (jnp.zeros((1, D), jnp.float32)
           .at[0, 0:5].set(bt1).at[0, 5:10].set(bt2))
    wpst = (jnp.zeros((D, D), jnp.float32)
            .at[0, 0:5].set(Wpmi[0]).at[1, 5:10].set(Wsign[0]))

    blk = 2048
    grid = (B // blk,)
    row_spec = pl.BlockSpec((blk, D), lambda i: (i, 0))
    w_spec = pl.BlockSpec((D, D), lambda i: (0, 0))
    b_spec = pl.BlockSpec((1, D), lambda i: (0, 0))
    s_spec = pl.BlockSpec(memory_space=pltpu.SMEM)
    o_spec = pl.BlockSpec((blk,), lambda i: (i,))
    o1, o2 = pl.pallas_call(
        _mlp_body,
        grid=grid,
        in_specs=[row_spec, w_spec, b_spec, w_spec, b_spec, w_spec,
                  s_spec, s_spec],
        out_specs=[o_spec, o_spec],
        out_shape=[jax.ShapeDtypeStruct((B,), jnp.float32),
                   jax.ShapeDtypeStruct((B,), jnp.float32)],
    )(prod, w1t, b1p, wtt, btt, wpst, bpmi, bsign)
    return (o1.reshape(B, 1), o2.reshape(B, 1))
